# baseline (speedup 1.0000x reference)
import functools

import jax
import jax.numpy as jnp
from jax import lax
from jax.experimental import pallas as pl
from jax.experimental.pallas import tpu as pltpu
from jax.experimental.pallas import tpu_sc as plsc

NC = 2
NS = 16
EB = 88
KB = 8


def _vector_mesh():
    return plsc.VectorSubcoreMesh(core_axis_name="c", subcore_axis_name="s")


def _deg_pass(dst3, ones_hbm, zeros_hbm, npad):
    nb = dst3.shape[1]
    rows_per = npad // NS

    @functools.partial(
        pl.kernel,
        out_type=jax.ShapeDtypeStruct((NC, npad, 128), jnp.float32),
        mesh=_vector_mesh(),
        scratch_types=[
            pltpu.VMEM_SHARED((npad, 128), jnp.float32),
            pltpu.VMEM((nb, EB), jnp.int32),
            pltpu.VMEM((EB, 128), jnp.float32),
        ],
    )
    def k(dst_hbm, ones_h, zeros_h, out_hbm, acc, idst, buf):
        core = lax.axis_index("c")
        sub = lax.axis_index("s")
        wid = core * NS + sub
        r0 = sub * rows_per
        pltpu.sync_copy(dst_hbm.at[wid], idst)
        pltpu.sync_copy(ones_h, buf)
        pltpu.sync_copy(zeros_h.at[pl.ds(r0, rows_per)], acc.at[pl.ds(r0, rows_per)])
        plsc.subcore_barrier()

        @pl.loop(0, nb)
        def _(j):
            pltpu.sync_copy(buf, acc.at[idst.at[j]], add=True)

        plsc.subcore_barrier()

        @pl.when(core == 0)
        def _():
            pltpu.sync_copy(acc.at[pl.ds(r0, rows_per)],
                            out_hbm.at[0].at[pl.ds(r0, rows_per)])

        @pl.when(core == 1)
        def _():
            pltpu.sync_copy(acc.at[pl.ds(r0, rows_per)],
                            out_hbm.at[1].at[pl.ds(r0, rows_per)])

    return k(dst3, ones_hbm, zeros_hbm)


def _edge_pass(tab, src3, dst3, n, npad):
    nb = src3.shape[1]
    rows_a = ((n // NS) + 7) & ~7
    rows_last = n - rows_a * (NS - 1)

    @functools.partial(
        pl.kernel,
        out_type=jax.ShapeDtypeStruct((NC, n, 128), jnp.float32),
        mesh=_vector_mesh(),
        scratch_types=[
            pltpu.VMEM_SHARED((npad, 128), jnp.float32),
            pltpu.VMEM((KB, EB), jnp.int32),
            pltpu.VMEM((KB, EB), jnp.int32),
            pltpu.VMEM((EB, 128), jnp.float32),
            pltpu.VMEM((EB, 128), jnp.float32),
            pltpu.VMEM((EB, 128), jnp.float32),
            pltpu.VMEM((EB, 128), jnp.float32),
            pltpu.SemaphoreType.DMA,
            pltpu.SemaphoreType.DMA,
            pltpu.SemaphoreType.DMA,
            pltpu.SemaphoreType.DMA,
            pltpu.SemaphoreType.DMA,
            pltpu.SemaphoreType.DMA,
            pltpu.SemaphoreType.DMA,
            pltpu.SemaphoreType.DMA,
        ],
    )
    def k(tab_hbm, src_hbm, dst_hbm, out_hbm, acc, isrc, idst,
          buf0, buf1, buf2, buf3, g0, g1, g2, g3, s0, s1, s2, s3):
        core = lax.axis_index("c")
        sub = lax.axis_index("s")
        r0 = sub * rows_a

        def run(tab_c, out_c):
            @pl.when(sub < NS - 1)
            def _():
                pltpu.sync_copy(tab_c.at[pl.ds(r0, rows_a)],
                                acc.at[pl.ds(r0, rows_a)])

            @pl.when(sub == NS - 1)
            def _():
                pltpu.sync_copy(tab_c.at[pl.ds(rows_a * (NS - 1), rows_last)],
                                acc.at[pl.ds(rows_a * (NS - 1), rows_last)])

            plsc.subcore_barrier()

            @pl.loop(0, nb, step=KB)
            def _(b):
                pltpu.sync_copy(src_hbm.at[sub].at[pl.ds(b, KB)], isrc)
                pltpu.sync_copy(dst_hbm.at[sub].at[pl.ds(b, KB)], idst)
                slots = ((buf0, g0, s0), (buf1, g1, s1),
                         (buf2, g2, s2), (buf3, g3, s3))
                nd = len(slots)
                for j in range(min(nd, KB)):
                    buf, g, _ = slots[j % nd]
                    pltpu.async_copy(tab_c.at[isrc.at[j]], buf, g)
                for j0 in range(0, KB, nd):
                    hi = min(j0 + nd, KB)
                    for j in range(j0, hi):
                        buf, g, s = slots[j % nd]
                        pltpu.make_async_copy(
                            tab_c.at[isrc.at[j]], buf, g).wait()
                        pltpu.async_copy(buf, acc.at[idst.at[j]], s, add=True)
                    for j in range(j0, hi):
                        buf, g, s = slots[j % nd]
                        pltpu.make_async_copy(
                            buf, acc.at[idst.at[j]], s).wait()
                        if j + nd < KB:
                            pltpu.async_copy(
                                tab_c.at[isrc.at[j + nd]], buf, g)

            plsc.subcore_barrier()

            @pl.when(sub < NS - 1)
            def _():
                pltpu.sync_copy(acc.at[pl.ds(r0, rows_a)],
                                out_c.at[pl.ds(r0, rows_a)])

            @pl.when(sub == NS - 1)
            def _():
                pltpu.sync_copy(acc.at[pl.ds(rows_a * (NS - 1), rows_last)],
                                out_c.at[pl.ds(rows_a * (NS - 1), rows_last)])

        @pl.when(core == 0)
        def _():
            run(tab_hbm.at[0], out_hbm.at[0])

        @pl.when(core == 1)
        def _():
            run(tab_hbm.at[1], out_hbm.at[1])

    return k(tab, src3, dst3)


def _dinv_tile(degp_blk):
    deg = degp_blk[0, :, 0:1] + degp_blk[1, :, 0:1] + 1.0
    return lax.rsqrt(deg)


def _mm1(x, W1, tm):
    m, kdim = x.shape
    odim = W1.shape[1]

    def body(x_ref, w_ref, o_ref):
        o_ref[...] = lax.dot_general(
            x_ref[...], w_ref[...], (((1,), (0,)), ((), ())),
            precision=lax.Precision.HIGHEST,
            preferred_element_type=jnp.float32)

    return pl.pallas_call(
        body,
        grid=(m // tm,),
        in_specs=[pl.BlockSpec((tm, kdim), lambda i: (i, 0)),
                  pl.BlockSpec((kdim, odim), lambda i: (0, 0))],
        out_specs=pl.BlockSpec((tm, odim), lambda i: (i, 0)),
        out_shape=jax.ShapeDtypeStruct((m, odim), jnp.float32),
    )(x, W1)


def _prescale1(degp, h1, n, npad, tm):
    def body(d_ref, h_ref, o_ref):
        dinv = _dinv_tile(d_ref[...])
        hp = dinv * h_ref[...]
        o_ref[0] = hp[:, :128]
        o_ref[1] = hp[:, 128:]

    return pl.pallas_call(
        body,
        grid=(n // tm,),
        in_specs=[pl.BlockSpec((NC, tm, 128), lambda i: (0, i, 0)),
                  pl.BlockSpec((tm, 256), lambda i: (i, 0))],
        out_specs=pl.BlockSpec((NC, tm, 128), lambda i: (0, i, 0)),
        out_shape=jax.ShapeDtypeStruct((NC, n, 128), jnp.float32),
    )(degp, h1)


def _mid(acc1, degp, b1r, wcat, n, tm):
    def body(a_ref, d_ref, b_ref, w_ref, o_ref):
        dinv = _dinv_tile(d_ref[...])
        a = a_ref[...]
        pre = dinv * jnp.concatenate([a[0], a[1]], axis=1) + b_ref[...]
        hidden = jnp.maximum(pre, 0.0)
        hc = lax.dot_general(
            hidden, w_ref[...], (((1,), (0,)), ((), ())),
            precision=lax.Precision.HIGHEST,
            preferred_element_type=jnp.float32)
        o_ref[0] = dinv * hc[:, :128]
        o_ref[1] = dinv * hc[:, 128:]

    return pl.pallas_call(
        body,
        grid=(n // tm,),
        in_specs=[pl.BlockSpec((NC, tm, 128), lambda i: (0, i, 0)),
                  pl.BlockSpec((NC, tm, 128), lambda i: (0, i, 0)),
                  pl.BlockSpec((1, 256), lambda i: (0, 0)),
                  pl.BlockSpec((256, 256), lambda i: (0, 0))],
        out_specs=pl.BlockSpec((NC, tm, 128), lambda i: (0, i, 0)),
        out_shape=jax.ShapeDtypeStruct((NC, n, 128), jnp.float32),
    )(acc1, degp, b1r, wcat)


def _final(acc2, degp, bmr, bvr, wl, blr, noise, n, tm):
    def body(a_ref, d_ref, bm_ref, bv_ref, wl_ref, bl_ref, nz_ref,
             mean_ref, lv_ref, z_ref):
        dinv = _dinv_tile(d_ref[...])
        a = a_ref[...]
        mean = dinv * a[0] + bm_ref[...]
        pv = jnp.maximum(dinv * a[1] + bv_ref[...], 0.0)
        lv = jnp.tanh(
            lax.dot_general(pv, wl_ref[...], (((1,), (0,)), ((), ())),
                            precision=lax.Precision.HIGHEST,
                            preferred_element_type=jnp.float32)
            + bl_ref[...])
        mean_ref[...] = mean
        lv_ref[...] = lv
        z_ref[...] = nz_ref[...] * jnp.exp(0.5 * lv) + mean

    out_sds = jax.ShapeDtypeStruct((n, 128), jnp.float32)
    return pl.pallas_call(
        body,
        grid=(n // tm,),
        in_specs=[pl.BlockSpec((NC, tm, 128), lambda i: (0, i, 0)),
                  pl.BlockSpec((NC, tm, 128), lambda i: (0, i, 0)),
                  pl.BlockSpec((1, 128), lambda i: (0, 0)),
                  pl.BlockSpec((1, 128), lambda i: (0, 0)),
                  pl.BlockSpec((128, 128), lambda i: (0, 0)),
                  pl.BlockSpec((1, 128), lambda i: (0, 0)),
                  pl.BlockSpec((tm, 128), lambda i: (i, 0))],
        out_specs=[pl.BlockSpec((tm, 128), lambda i: (i, 0))] * 3,
        out_shape=[out_sds, out_sds, out_sds],
    )(acc2, degp, bmr, bvr, wl, blr, noise)


def kernel(x, edge_index, W1, b1, Wm, bm, Wv, bv, Wl, bl, noise):
    n = x.shape[0]
    e = edge_index.shape[1]
    npad = (n // 128 + 1) * 128
    tm = 1000

    src = edge_index[0]
    dst = edge_index[1]
    pad = (-e) % (NS * EB * KB)
    if pad:
        src = jnp.concatenate([src, jnp.zeros((pad,), src.dtype)])
        dst = jnp.concatenate(
            [dst, n + jnp.arange(pad, dtype=dst.dtype) % (npad - n)])
    src16 = src.reshape(NS, -1, EB)
    dst16 = dst.reshape(NS, -1, EB)
    dst32 = dst.reshape(NC * NS, -1, EB)

    ones128 = jnp.ones((EB, 128), jnp.float32)
    zeros128 = jnp.zeros((npad, 128), jnp.float32)

    degp = _deg_pass(dst32, ones128, zeros128, npad)
    h1 = _mm1(x, W1, tm)
    tab1 = _prescale1(degp, h1, n, npad, tm)
    acc1 = _edge_pass(tab1, src16, dst16, n, npad)
    wcat = jnp.concatenate([Wm, Wv], axis=1)
    tab2 = _mid(acc1, degp, b1.reshape(1, -1), wcat, n, tm)
    acc2 = _edge_pass(tab2, src16, dst16, n, npad)
    mean, logvar, z = _final(acc2, degp, bm.reshape(1, -1), bv.reshape(1, -1),
                             Wl, bl.reshape(1, -1), noise, n, tm)
    return (mean, logvar, z)


# speedup vs baseline: 1.7057x; 1.7057x over previous
import functools

import jax
import jax.numpy as jnp
from jax import lax
from jax.experimental import pallas as pl
from jax.experimental.pallas import tpu as pltpu
from jax.experimental.pallas import tpu_sc as plsc

NC = 2
NS = 16
EB = 120
KB = 12


def _vector_mesh():
    return plsc.VectorSubcoreMesh(core_axis_name="c", subcore_axis_name="s")


def _deg_pass(dst3, ones_hbm, zeros_hbm, npad):
    nb = dst3.shape[1]
    rows_per = npad // NS

    @functools.partial(
        pl.kernel,
        out_type=jax.ShapeDtypeStruct((NC, npad, 128), jnp.float32),
        mesh=_vector_mesh(),
        scratch_types=[
            pltpu.VMEM_SHARED((npad, 128), jnp.float32),
            pltpu.VMEM((nb, EB), jnp.int32),
            pltpu.VMEM((EB, 128), jnp.float32),
        ],
    )
    def k(dst_hbm, ones_h, zeros_h, out_hbm, acc, idst, buf):
        core = lax.axis_index("c")
        sub = lax.axis_index("s")
        wid = core * NS + sub
        r0 = sub * rows_per
        pltpu.sync_copy(dst_hbm.at[wid], idst)
        pltpu.sync_copy(ones_h, buf)
        pltpu.sync_copy(zeros_h.at[pl.ds(r0, rows_per)], acc.at[pl.ds(r0, rows_per)])
        plsc.subcore_barrier()

        @pl.loop(0, nb)
        def _(j):
            pltpu.sync_copy(buf, acc.at[idst.at[j]], add=True)

        plsc.subcore_barrier()

        @pl.when(core == 0)
        def _():
            pltpu.sync_copy(acc.at[pl.ds(r0, rows_per)],
                            out_hbm.at[0].at[pl.ds(r0, rows_per)])

        @pl.when(core == 1)
        def _():
            pltpu.sync_copy(acc.at[pl.ds(r0, rows_per)],
                            out_hbm.at[1].at[pl.ds(r0, rows_per)])

    return k(dst3, ones_hbm, zeros_hbm)


def _edge_pass(tab, src3, dst3, n, npad):
    nchunks = src3.shape[1]
    rows_a = ((n // NS) + 7) & ~7
    rows_last = n - rows_a * (NS - 1)

    @functools.partial(
        pl.kernel,
        out_type=jax.ShapeDtypeStruct((NC, n, 128), jnp.float32),
        mesh=_vector_mesh(),
        scratch_types=[
            pltpu.VMEM_SHARED((npad, 128), jnp.float32),
            pltpu.VMEM((KB, EB), jnp.int32),
            pltpu.VMEM((KB, EB), jnp.int32),
            pltpu.VMEM((EB, 128), jnp.float32),
            pltpu.VMEM((EB, 128), jnp.float32),
            pltpu.VMEM((EB, 128), jnp.float32),
            pltpu.SemaphoreType.DMA,
            pltpu.SemaphoreType.DMA,
            pltpu.SemaphoreType.DMA,
            pltpu.SemaphoreType.DMA,
            pltpu.SemaphoreType.DMA,
            pltpu.SemaphoreType.DMA,
        ],
    )
    def k(tab_hbm, src_hbm, dst_hbm, out_hbm, acc, isrc, idst,
          buf0, buf1, buf2, g0, g1, g2, s0, s1, s2):
        core = lax.axis_index("c")
        sub = lax.axis_index("s")
        r0 = sub * rows_a

        def run(tab_c, out_c):
            @pl.when(sub < NS - 1)
            def _():
                pltpu.sync_copy(tab_c.at[pl.ds(r0, rows_a)],
                                acc.at[pl.ds(r0, rows_a)])

            @pl.when(sub == NS - 1)
            def _():
                pltpu.sync_copy(tab_c.at[pl.ds(rows_a * (NS - 1), rows_last)],
                                acc.at[pl.ds(rows_a * (NS - 1), rows_last)])

            plsc.subcore_barrier()

            @pl.loop(0, nchunks)
            def _(c):
                pltpu.sync_copy(src_hbm.at[sub].at[c], isrc)
                pltpu.sync_copy(dst_hbm.at[sub].at[c], idst)
                slots = ((buf0, g0, s0), (buf1, g1, s1), (buf2, g2, s2))
                nd = len(slots)
                for j in range(min(nd, KB)):
                    buf, g, _ = slots[j % nd]
                    pltpu.async_copy(tab_c.at[isrc.at[j]], buf, g)
                for j0 in range(0, KB, nd):
                    hi = min(j0 + nd, KB)
                    for j in range(j0, hi):
                        buf, g, s = slots[j % nd]
                        pltpu.make_async_copy(
                            tab_c.at[isrc.at[j]], buf, g).wait()
                        pltpu.async_copy(buf, acc.at[idst.at[j]], s, add=True)
                    for j in range(j0, hi):
                        buf, g, s = slots[j % nd]
                        pltpu.make_async_copy(
                            buf, acc.at[idst.at[j]], s).wait()
                        if j + nd < KB:
                            pltpu.async_copy(
                                tab_c.at[isrc.at[j + nd]], buf, g)

            plsc.subcore_barrier()

            @pl.when(sub < NS - 1)
            def _():
                pltpu.sync_copy(acc.at[pl.ds(r0, rows_a)],
                                out_c.at[pl.ds(r0, rows_a)])

            @pl.when(sub == NS - 1)
            def _():
                pltpu.sync_copy(acc.at[pl.ds(rows_a * (NS - 1), rows_last)],
                                out_c.at[pl.ds(rows_a * (NS - 1), rows_last)])

        @pl.when(core == 0)
        def _():
            run(tab_hbm.at[0], out_hbm.at[0])

        @pl.when(core == 1)
        def _():
            run(tab_hbm.at[1], out_hbm.at[1])

    return k(tab, src3, dst3)


def _dinv_tile(degp_blk):
    deg = degp_blk[0, :, 0:1] + degp_blk[1, :, 0:1] + 1.0
    return lax.rsqrt(deg)


def _mm1(x, W1, tm):
    m, kdim = x.shape
    odim = W1.shape[1]

    def body(x_ref, w_ref, o_ref):
        o_ref[...] = lax.dot_general(
            x_ref[...], w_ref[...], (((1,), (0,)), ((), ())),
            precision=lax.Precision.HIGHEST,
            preferred_element_type=jnp.float32)

    return pl.pallas_call(
        body,
        grid=(m // tm,),
        in_specs=[pl.BlockSpec((tm, kdim), lambda i: (i, 0)),
                  pl.BlockSpec((kdim, odim), lambda i: (0, 0))],
        out_specs=pl.BlockSpec((tm, odim), lambda i: (i, 0)),
        out_shape=jax.ShapeDtypeStruct((m, odim), jnp.float32),
    )(x, W1)


def _prescale1(degp, h1, n, npad, tm):
    def body(d_ref, h_ref, o_ref):
        dinv = _dinv_tile(d_ref[...])
        hp = dinv * h_ref[...]
        o_ref[0] = hp[:, :128]
        o_ref[1] = hp[:, 128:]

    return pl.pallas_call(
        body,
        grid=(n // tm,),
        in_specs=[pl.BlockSpec((NC, tm, 128), lambda i: (0, i, 0)),
                  pl.BlockSpec((tm, 256), lambda i: (i, 0))],
        out_specs=pl.BlockSpec((NC, tm, 128), lambda i: (0, i, 0)),
        out_shape=jax.ShapeDtypeStruct((NC, n, 128), jnp.float32),
    )(degp, h1)


def _mid(acc1, degp, b1r, wcat, n, tm):
    def body(a_ref, d_ref, b_ref, w_ref, o_ref):
        dinv = _dinv_tile(d_ref[...])
        a = a_ref[...]
        pre = dinv * jnp.concatenate([a[0], a[1]], axis=1) + b_ref[...]
        hidden = jnp.maximum(pre, 0.0)
        hc = lax.dot_general(
            hidden, w_ref[...], (((1,), (0,)), ((), ())),
            precision=lax.Precision.HIGHEST,
            preferred_element_type=jnp.float32)
        o_ref[0] = dinv * hc[:, :128]
        o_ref[1] = dinv * hc[:, 128:]

    return pl.pallas_call(
        body,
        grid=(n // tm,),
        in_specs=[pl.BlockSpec((NC, tm, 128), lambda i: (0, i, 0)),
                  pl.BlockSpec((NC, tm, 128), lambda i: (0, i, 0)),
                  pl.BlockSpec((1, 256), lambda i: (0, 0)),
                  pl.BlockSpec((256, 256), lambda i: (0, 0))],
        out_specs=pl.BlockSpec((NC, tm, 128), lambda i: (0, i, 0)),
        out_shape=jax.ShapeDtypeStruct((NC, n, 128), jnp.float32),
    )(acc1, degp, b1r, wcat)


def _final(acc2, degp, bmr, bvr, wl, blr, noise, n, tm):
    def body(a_ref, d_ref, bm_ref, bv_ref, wl_ref, bl_ref, nz_ref,
             mean_ref, lv_ref, z_ref):
        dinv = _dinv_tile(d_ref[...])
        a = a_ref[...]
        mean = dinv * a[0] + bm_ref[...]
        pv = jnp.maximum(dinv * a[1] + bv_ref[...], 0.0)
        lv = jnp.tanh(
            lax.dot_general(pv, wl_ref[...], (((1,), (0,)), ((), ())),
                            precision=lax.Precision.HIGHEST,
                            preferred_element_type=jnp.float32)
            + bl_ref[...])
        mean_ref[...] = mean
        lv_ref[...] = lv
        z_ref[...] = nz_ref[...] * jnp.exp(0.5 * lv) + mean

    out_sds = jax.ShapeDtypeStruct((n, 128), jnp.float32)
    return pl.pallas_call(
        body,
        grid=(n // tm,),
        in_specs=[pl.BlockSpec((NC, tm, 128), lambda i: (0, i, 0)),
                  pl.BlockSpec((NC, tm, 128), lambda i: (0, i, 0)),
                  pl.BlockSpec((1, 128), lambda i: (0, 0)),
                  pl.BlockSpec((1, 128), lambda i: (0, 0)),
                  pl.BlockSpec((128, 128), lambda i: (0, 0)),
                  pl.BlockSpec((1, 128), lambda i: (0, 0)),
                  pl.BlockSpec((tm, 128), lambda i: (i, 0))],
        out_specs=[pl.BlockSpec((tm, 128), lambda i: (i, 0))] * 3,
        out_shape=[out_sds, out_sds, out_sds],
    )(acc2, degp, bmr, bvr, wl, blr, noise)


def kernel(x, edge_index, W1, b1, Wm, bm, Wv, bv, Wl, bl, noise):
    n = x.shape[0]
    e = edge_index.shape[1]
    npad = (n // 128 + 1) * 128
    tm = 1000

    src = edge_index[0]
    dst = edge_index[1]
    pad = (-e) % (NS * EB * KB)
    if pad:
        src = jnp.concatenate([src, jnp.zeros((pad,), src.dtype)])
        dst = jnp.concatenate(
            [dst, n + jnp.arange(pad, dtype=dst.dtype) % (npad - n)])
    src16 = src.reshape(NS, -1, KB, EB)
    dst16 = dst.reshape(NS, -1, KB, EB)
    dst32 = dst.reshape(NC * NS, -1, EB)

    ones128 = jnp.ones((EB, 128), jnp.float32)
    zeros128 = jnp.zeros((npad, 128), jnp.float32)

    degp = _deg_pass(dst32, ones128, zeros128, npad)
    h1 = _mm1(x, W1, tm)
    tab1 = _prescale1(degp, h1, n, npad, tm)
    acc1 = _edge_pass(tab1, src16, dst16, n, npad)
    wcat = jnp.concatenate([Wm, Wv], axis=1)
    tab2 = _mid(acc1, degp, b1.reshape(1, -1), wcat, n, tm)
    acc2 = _edge_pass(tab2, src16, dst16, n, npad)
    mean, logvar, z = _final(acc2, degp, bm.reshape(1, -1), bv.reshape(1, -1),
                             Wl, bl.reshape(1, -1), noise, n, tm)
    return (mean, logvar, z)
